# baseline (speedup 1.0000x reference)
import jax
import jax.numpy as jnp
from jax.experimental import pallas as pl
from jax.experimental.pallas import tpu as pltpu

_N = 2048
_D = 256
_K = 16
_RB = 256
_NRB = _N // _RB


def _split_bf16(a):
    hi = a.astype(jnp.bfloat16)
    lo = (a - hi.astype(jnp.float32)).astype(jnp.bfloat16)
    return hi, lo


def _dot3(a, b, dims):
    ah, al = _split_bf16(a)
    bh, bl = _split_bf16(b)
    dot = lambda u, v: jax.lax.dot_general(
        u, v, dims, preferred_element_type=jnp.float32)
    return dot(ah, bh) + (dot(ah, bl) + dot(al, bh))


def _mm(a, b):
    return _dot3(a, b, (((1,), (0,)), ((), ())))


def _body(x_ref, pos_ref, post_ref, Wq_ref, bq_ref, Wk_ref, bk_ref,
          Wv_ref, bv_ref, Wpe_ref, bpe_ref, Wpd_ref, wa_ref, Wob_ref, bo_ref,
          out_ref, Cb_scr, Vb_scr, Qb_scr, U_scr):
    r = pl.program_id(1)

    @pl.when(r == 0)
    def _proj():
        x = x_ref[0]
        pos = pos_ref[0]
        pe = (pos[:, 0:1] * Wpe_ref[0:1, :]
              + pos[:, 1:2] * Wpe_ref[1:2, :]
              + pos[:, 2:3] * Wpe_ref[2:3, :]) + bpe_ref[...]
        q = _mm(x, Wq_ref[...]) + bq_ref[...]
        k = _mm(x, Wk_ref[...]) + bk_ref[...]
        v = _mm(x, Wv_ref[...]) + bv_ref[...]
        qw = (q + pe) * wa_ref[...] * jnp.float32(1.0 / 16.0)
        Cb_scr[...] = (k + pe).astype(jnp.bfloat16)
        Vb_scr[...] = v.astype(jnp.bfloat16)
        Qb_scr[...] = qw.astype(jnp.bfloat16)
        u0 = jnp.sum(qw * Wpd_ref[0:1, :], axis=1, keepdims=True)
        u1 = jnp.sum(qw * Wpd_ref[1:2, :], axis=1, keepdims=True)
        u2 = jnp.sum(qw * Wpd_ref[2:3, :], axis=1, keepdims=True)
        U_scr[...] = jnp.concatenate([u0, u1, u2], axis=1)

    rows = pl.ds(r * _RB, _RB)
    posb = pos_ref[0, rows]
    post = post_ref[0]
    sq = (post[0:1, :] * post[0:1, :]
          + post[1:2, :] * post[1:2, :]
          + post[2:3, :] * post[2:3, :])
    m2 = posb * jnp.float32(-2.0)
    dist = sq + _dot3(m2, post, (((1,), (0,)), ((), ())))

    inf = jnp.float32(jnp.inf)
    a = dist[:, :_N // 2]
    b = dist[:, _N // 2:]
    cur = jnp.minimum(a, b)
    shadow = jnp.maximum(a, b)
    t = jnp.float32(0.0)
    for _ in range(_K):
        t = jnp.min(cur, axis=1, keepdims=True)
        sel = cur <= t
        cur = jnp.where(sel, shadow, cur)
        shadow = jnp.where(sel, inf, shadow)
    mask = dist <= t

    dot = lambda u, v, dims: jax.lax.dot_general(
        u, v, dims, preferred_element_type=jnp.float32)
    dT = (((1,), (1,)), ((), ()))
    d0 = (((1,), (0,)), ((), ()))
    logits = dot(Qb_scr[rows], Cb_scr[...], dT)
    ub = U_scr[rows]
    logits = (logits
              + ub[:, 0:1] * post[0:1, :]
              + ub[:, 1:2] * post[1:2, :]
              + ub[:, 2:3] * post[2:3, :])
    logits = jnp.where(mask, logits, jnp.float32(-1e30))
    e = jnp.exp(logits)
    s = jnp.sum(e, axis=1, keepdims=True)
    eb = e.astype(jnp.bfloat16)
    acc = dot(eb, Vb_scr[...], d0)
    outp = acc / s
    ob = outp.astype(jnp.bfloat16)
    o2 = dot(ob, Wob_ref[...], d0) + bo_ref[...]
    out_ref[0] = x_ref[0, rows] + jax.nn.gelu(o2)


def kernel(x, pos, Wq, bq, Wk, bk, Wv, bv, Wpe, bpe, Wpd, bpd, Wa, ba, Wo, bo):
    B, S, N, D = x.shape
    x2 = x.reshape(B, N, D)
    pos2 = pos.reshape(B, N, 3)
    pos_t = pos2.transpose(0, 2, 1)
    wa = Wa.reshape(1, D)
    Wob = Wo.astype(jnp.bfloat16)

    grid = (B, _NRB)
    full = lambda b, r: (b, 0, 0)
    blk = lambda b, r: (b, r, 0)
    wfull = lambda b, r: (0, 0)

    out = pl.pallas_call(
        _body,
        grid=grid,
        in_specs=[
            pl.BlockSpec((1, N, D), full),
            pl.BlockSpec((1, N, 3), full),
            pl.BlockSpec((1, 3, N), full),
            pl.BlockSpec((D, D), wfull),
            pl.BlockSpec((1, D), wfull),
            pl.BlockSpec((D, D), wfull),
            pl.BlockSpec((1, D), wfull),
            pl.BlockSpec((D, D), wfull),
            pl.BlockSpec((1, D), wfull),
            pl.BlockSpec((3, D), wfull),
            pl.BlockSpec((1, D), wfull),
            pl.BlockSpec((3, D), wfull),
            pl.BlockSpec((1, D), wfull),
            pl.BlockSpec((D, D), wfull),
            pl.BlockSpec((1, D), wfull),
        ],
        out_specs=pl.BlockSpec((1, _RB, D), blk),
        out_shape=jax.ShapeDtypeStruct((B, N, D), jnp.float32),
        scratch_shapes=[
            pltpu.VMEM((N, D), jnp.bfloat16),
            pltpu.VMEM((N, D), jnp.bfloat16),
            pltpu.VMEM((N, D), jnp.bfloat16),
            pltpu.VMEM((N, 3), jnp.float32),
        ],
        compiler_params=pltpu.CompilerParams(
            dimension_semantics=("arbitrary", "arbitrary")),
    )(x2, pos2, pos_t, Wq, bq.reshape(1, D), Wk, bk.reshape(1, D),
      Wv, bv.reshape(1, D), Wpe, bpe.reshape(1, D), Wpd, wa,
      Wob, bo.reshape(1, D))

    return out.reshape(B, S, N, D)


# speedup vs baseline: 1.1370x; 1.1370x over previous
import jax
import jax.numpy as jnp
from jax.experimental import pallas as pl
from jax.experimental.pallas import tpu as pltpu

_N = 2048
_D = 256
_K = 16
_RB = 256
_NRB = _N // _RB


def _split_bf16(a):
    hi = a.astype(jnp.bfloat16)
    lo = (a - hi.astype(jnp.float32)).astype(jnp.bfloat16)
    return hi, lo


def _dot3(a, b, dims):
    ah, al = _split_bf16(a)
    bh, bl = _split_bf16(b)
    dot = lambda u, v: jax.lax.dot_general(
        u, v, dims, preferred_element_type=jnp.float32)
    return dot(ah, bh) + (dot(ah, bl) + dot(al, bh))


def _mm(a, b):
    return _dot3(a, b, (((1,), (0,)), ((), ())))


def _body(x_ref, pos_ref, post_ref, Wq_ref, bq_ref, Wk_ref, bk_ref,
          Wv_ref, bv_ref, Wpe_ref, bpe_ref, Wpd_ref, wa_ref, Wob_ref, bo_ref,
          out_ref, Cb_scr, Vb_scr, Qb_scr, U_scr):
    r = pl.program_id(1)

    @pl.when(r == 0)
    def _proj():
        x = x_ref[0]
        pos = pos_ref[0]
        pe = (pos[:, 0:1] * Wpe_ref[0:1, :]
              + pos[:, 1:2] * Wpe_ref[1:2, :]
              + pos[:, 2:3] * Wpe_ref[2:3, :]) + bpe_ref[...]
        q = _mm(x, Wq_ref[...]) + bq_ref[...]
        k = _mm(x, Wk_ref[...]) + bk_ref[...]
        v = _mm(x, Wv_ref[...]) + bv_ref[...]
        qw = (q + pe) * wa_ref[...] * jnp.float32(1.0 / 16.0)
        Cb_scr[...] = (k + pe).astype(jnp.bfloat16)
        Vb_scr[...] = v.astype(jnp.bfloat16)
        Qb_scr[...] = qw.astype(jnp.bfloat16)
        u0 = jnp.sum(qw * Wpd_ref[0:1, :], axis=1, keepdims=True)
        u1 = jnp.sum(qw * Wpd_ref[1:2, :], axis=1, keepdims=True)
        u2 = jnp.sum(qw * Wpd_ref[2:3, :], axis=1, keepdims=True)
        U_scr[...] = jnp.concatenate([u0, u1, u2], axis=1)

    rows = pl.ds(r * _RB, _RB)
    posb = pos_ref[0, rows]
    post = post_ref[0]
    sq = (post[0:1, :] * post[0:1, :]
          + post[1:2, :] * post[1:2, :]
          + post[2:3, :] * post[2:3, :])
    m2 = posb * jnp.float32(-2.0)
    dist = (sq
            + m2[:, 0:1] * post[0:1, :]
            + m2[:, 1:2] * post[1:2, :]
            + m2[:, 2:3] * post[2:3, :])

    inf = jnp.float32(jnp.inf)
    a = dist[:, :_N // 2]
    b = dist[:, _N // 2:]
    cur = jnp.minimum(a, b)
    shadow = jnp.maximum(a, b)
    t = jnp.float32(0.0)
    for _ in range(_K):
        t = jnp.min(cur, axis=1, keepdims=True)
        sel = cur <= t
        cur = jnp.where(sel, jnp.where(cur >= shadow, inf, shadow), cur)
    mask = dist <= t

    dot = lambda u, v, dims: jax.lax.dot_general(
        u, v, dims, preferred_element_type=jnp.float32)
    dT = (((1,), (1,)), ((), ()))
    d0 = (((1,), (0,)), ((), ()))
    logits = dot(Qb_scr[rows], Cb_scr[...], dT)
    ub = U_scr[rows]
    logits = (logits
              + ub[:, 0:1] * post[0:1, :]
              + ub[:, 1:2] * post[1:2, :]
              + ub[:, 2:3] * post[2:3, :])
    logits = jnp.where(mask, logits, jnp.float32(-1e30))
    e = jnp.exp(logits)
    s = jnp.sum(e, axis=1, keepdims=True)
    eb = e.astype(jnp.bfloat16)
    acc = dot(eb, Vb_scr[...], d0)
    outp = acc / s
    ob = outp.astype(jnp.bfloat16)
    o2 = dot(ob, Wob_ref[...], d0) + bo_ref[...]
    out_ref[0] = x_ref[0, rows] + jax.nn.gelu(o2)


def kernel(x, pos, Wq, bq, Wk, bk, Wv, bv, Wpe, bpe, Wpd, bpd, Wa, ba, Wo, bo):
    B, S, N, D = x.shape
    x2 = x.reshape(B, N, D)
    pos2 = pos.reshape(B, N, 3)
    pos_t = pos2.transpose(0, 2, 1)
    wa = Wa.reshape(1, D)
    Wob = Wo.astype(jnp.bfloat16)

    grid = (B, _NRB)
    full = lambda b, r: (b, 0, 0)
    blk = lambda b, r: (b, r, 0)
    wfull = lambda b, r: (0, 0)

    out = pl.pallas_call(
        _body,
        grid=grid,
        in_specs=[
            pl.BlockSpec((1, N, D), full),
            pl.BlockSpec((1, N, 3), full),
            pl.BlockSpec((1, 3, N), full),
            pl.BlockSpec((D, D), wfull),
            pl.BlockSpec((1, D), wfull),
            pl.BlockSpec((D, D), wfull),
            pl.BlockSpec((1, D), wfull),
            pl.BlockSpec((D, D), wfull),
            pl.BlockSpec((1, D), wfull),
            pl.BlockSpec((3, D), wfull),
            pl.BlockSpec((1, D), wfull),
            pl.BlockSpec((3, D), wfull),
            pl.BlockSpec((1, D), wfull),
            pl.BlockSpec((D, D), wfull),
            pl.BlockSpec((1, D), wfull),
        ],
        out_specs=pl.BlockSpec((1, _RB, D), blk),
        out_shape=jax.ShapeDtypeStruct((B, N, D), jnp.float32),
        scratch_shapes=[
            pltpu.VMEM((N, D), jnp.bfloat16),
            pltpu.VMEM((N, D), jnp.bfloat16),
            pltpu.VMEM((N, D), jnp.bfloat16),
            pltpu.VMEM((N, 3), jnp.float32),
        ],
        compiler_params=pltpu.CompilerParams(
            dimension_semantics=("arbitrary", "arbitrary")),
    )(x2, pos2, pos_t, Wq, bq.reshape(1, D), Wk, bk.reshape(1, D),
      Wv, bv.reshape(1, D), Wpe, bpe.reshape(1, D), Wpd, wa,
      Wob, bo.reshape(1, D))

    return out.reshape(B, S, N, D)
